# baseline (speedup 1.0000x reference)
import dataclasses
import functools

import jax
import jax.numpy as jnp
from jax import lax
from jax.experimental import pallas as pl
from jax.experimental.pallas import tpu as pltpu
from jax.experimental.pallas import tpu_sc as plsc

_H, _W = 240, 320
_N = _H * _W
_R = 9
_P = _R * _N
_V = 256
_VN = _V * _V * _V
_TROWS = _VN // 8
_BLK = 9600
_GRID = _P // _BLK

_NW = 32
_PTS_W = _P // _NW
_CH = 240
_NCHK = _PTS_W // _CH
_NGRP = _CH // 16
_FROW = 128


def _world_points(depth, extrinsics, intrinsics):
    b, h, w = depth.shape
    n = h * w
    xx, yy = jnp.meshgrid(jnp.arange(h, dtype=jnp.float32),
                          jnp.arange(w, dtype=jnp.float32), indexing="ij")
    xx = jnp.broadcast_to(xx.reshape(1, n, 1), (b, n, 1))
    yy = jnp.broadcast_to(yy.reshape(1, n, 1), (b, n, 1))
    zz = depth.reshape(b, n, 1)
    points_p = jnp.concatenate([yy * zz, xx * zz, zz], axis=2)
    intr_inv = jnp.linalg.inv(intrinsics.astype(jnp.float32))
    points_c = jnp.matmul(intr_inv, jnp.transpose(points_p, (0, 2, 1)))
    homog = jnp.ones((b, 1, n), dtype=points_c.dtype)
    points_c = jnp.concatenate([points_c, homog], axis=1)
    points_w = jnp.matmul(extrinsics[:3], points_c)
    points_w = jnp.transpose(points_w, (0, 2, 1))[:, :, :3]
    return points_w


def _rays(coords, eye, origin, resolution, n_points=4, bin_size=1.0):
    center_v = (coords - origin) / resolution
    eye_v = (eye - origin) / resolution
    direction = center_v - eye_v[:, None, :]
    norm = jnp.linalg.norm(direction, axis=2, keepdims=True)
    direction = direction / jnp.maximum(norm, 1e-12)
    points = [center_v]
    for i in range(1, n_points + 1):
        points.append(center_v + i * bin_size * direction)
        points.insert(0, center_v - i * bin_size * direction)
    return jnp.stack(points, axis=1)


def _trilinear_body(p_ref, w_ref, ix_ref, iy_ref, iz_ref, rid_ref, pz_ref,
                    wm_ref):
    pts = p_ref[...]

    def dimq(p):
        idx = jnp.floor(p)
        center = idx + 0.5
        neigh = jnp.sign(center - p)
        alpha = jnp.abs(p - center)
        c0, c1 = idx, idx + neigh
        v0 = (c0 >= 0.0) & (c0 < float(_V))
        v1 = (c1 >= 0.0) & (c1 < float(_V))
        cl = jnp.clip(jnp.minimum(c0, c1), 0.0, float(_V - 1))
        return c0, c1, 1.0 - alpha, alpha, v0, v1, cl

    x0, x1, wx0, wx1, vx0, vx1, clx = dimq(pts[0:1])
    y0, y1, wy0, wy1, vy0, vy1, cly = dimq(pts[1:2])
    z0, z1, wz0, wz1, vz0, vz1, clz = dimq(pts[2:3])

    s = lax.broadcasted_iota(jnp.int32, (8, _BLK), 0)
    bi, bj, bk = (s >> 2) & 1, (s >> 1) & 1, s & 1

    wx = jnp.where(bi == 0, wx0, wx1)
    wy = jnp.where(bj == 0, wy0, wy1)
    wz = jnp.where(bk == 0, wz0, wz1)
    w_ref[...] = (wx * wy) * wz
    ix_ref[...] = jnp.where(bi == 0, x0, x1)
    iy_ref[...] = jnp.where(bj == 0, y0, y1)
    iz_ref[...] = jnp.where(bk == 0, z0, z1)

    def slotw(pos, c0, c1, w0, w1, v0, v1):
        return (jnp.where((pos == c0) & v0, w0, 0.0)
                + jnp.where((pos == c1) & v1, w1, 0.0))

    ux = slotw(clx + bi.astype(jnp.float32), x0, x1, wx0, wx1, vx0, vx1)
    uy = slotw(cly + bj.astype(jnp.float32), y0, y1, wy0, wy1, vy0, vy1)
    uz = slotw(clz + bk.astype(jnp.float32), z0, z1, wz0, wz1, vz0, vz1)
    wm_ref[...] = (ux * uy) * uz

    s4 = lax.broadcasted_iota(jnp.int32, (4, _BLK), 0)
    a4, b4 = (s4 >> 1) & 1, s4 & 1
    xi = clx.astype(jnp.int32) + a4
    yi = cly.astype(jnp.int32) + b4
    zi = clz.astype(jnp.int32)
    rid_ref[...] = (xi * (_V * _V) + yi * _V + zi) >> 3
    pz_ref[...] = zi & 7


_trilinear_call = pl.pallas_call(
    _trilinear_body,
    grid=(_GRID,),
    in_specs=[pl.BlockSpec((3, _BLK), lambda i: (jnp.int32(0), i))],
    out_specs=[pl.BlockSpec((8, _BLK), lambda i: (jnp.int32(0), i))] * 4
    + [pl.BlockSpec((4, _BLK), lambda i: (jnp.int32(0), i)),
       pl.BlockSpec((1, _BLK), lambda i: (jnp.int32(0), i)),
       pl.BlockSpec((8, _BLK), lambda i: (jnp.int32(0), i))],
    out_shape=[
        jax.ShapeDtypeStruct((8, _P), jnp.float32),
        jax.ShapeDtypeStruct((8, _P), jnp.float32),
        jax.ShapeDtypeStruct((8, _P), jnp.float32),
        jax.ShapeDtypeStruct((8, _P), jnp.float32),
        jax.ShapeDtypeStruct((4, _P), jnp.int32),
        jax.ShapeDtypeStruct((1, _P), jnp.int32),
        jax.ShapeDtypeStruct((8, _P), jnp.float32),
    ],
)


@functools.lru_cache(maxsize=None)
def _make_gather():
    mesh = plsc.VectorSubcoreMesh(core_axis_name="c", subcore_axis_name="s")
    cp = pltpu.CompilerParams()
    for fld, val in (("needs_layout_passes", False),
                     ("use_tc_tiling_on_sc", False)):
        if fld in pltpu.CompilerParams.__dataclass_fields__:
            cp = dataclasses.replace(cp, **{fld: val})

    @functools.partial(
        pl.kernel,
        out_type=jax.ShapeDtypeStruct((16, _P), jnp.float32),
        mesh=mesh,
        compiler_params=cp,
        scratch_types=[pltpu.VMEM((_FROW,), jnp.int32),
                       pltpu.VMEM((4, _CH), jnp.int32),
                       pltpu.VMEM((_CH,), jnp.int32),
                       pltpu.VMEM((_CH, 32), jnp.float32),
                       pltpu.VMEM((_CH, 32), jnp.float32),
                       pltpu.VMEM((_CH, 32), jnp.float32),
                       pltpu.VMEM((_CH, 32), jnp.float32),
                       pltpu.VMEM((16, _CH), jnp.float32),
                       pltpu.SemaphoreType.DMA],
    )
    def _gather(tc_hbm, rid_hbm, pz_hbm, flags_hbm, g_hbm,
                fl_v, rid_v, pz_v, g0, g1, g2, g3, ob, sem):
        wid = lax.axis_index("s") * jnp.int32(2) + lax.axis_index("c")
        pt0 = wid * jnp.int32(_PTS_W)
        pltpu.sync_copy(flags_hbm.at[wid], fl_v)
        gbufs = (g0, g1, g2, g3)

        def _chunk(ci, carry):
            grp16 = (ci >> jnp.int32(4)) << jnp.int32(4)
            lane = ci & jnp.int32(15)
            fvec = fl_v[pl.ds(grp16, 16)]
            sel = jnp.where(lax.iota(jnp.int32, 16) == lane,
                            fvec, jnp.int32(0))
            pred = lax.reduce_max(sel, axes=(0,))

            @pl.when(pred != 0)
            def _do():
                off = pt0 + ci * jnp.int32(_CH)
                for c in range(4):
                    pltpu.sync_copy(rid_hbm.at[jnp.int32(c), pl.ds(off, _CH)],
                                    rid_v.at[jnp.int32(c)])
                pltpu.sync_copy(pz_hbm.at[pl.ds(off, _CH)], pz_v)
                cps = [pltpu.async_copy(tc_hbm.at[rid_v.at[jnp.int32(c)]],
                                        gbufs[c], sem)
                       for c in range(4)]
                for c in range(4):
                    cps[c].wait()
                for g in range(_NGRP):
                    rows = lax.iota(jnp.int32, 16) + jnp.int32(g * 16)
                    pvec = pz_v[pl.ds(g * 16, 16)]
                    for sl in range(8):
                        col = pvec + jnp.int32(sl & 1)
                        ob[jnp.int32(sl), pl.ds(g * 16, 16)] = (
                            plsc.load_gather(gbufs[sl >> 1], [rows, col]))
                        ob[jnp.int32(8 + sl), pl.ds(g * 16, 16)] = (
                            plsc.load_gather(gbufs[sl >> 1],
                                             [rows, col + jnp.int32(16)]))
                for sl in range(16):
                    pltpu.sync_copy(ob.at[jnp.int32(sl)],
                                    g_hbm.at[jnp.int32(sl), pl.ds(off, _CH)])

            return carry

        lax.fori_loop(jnp.int32(0), jnp.int32(_NCHK), _chunk, jnp.int32(0))

    return _gather


def _gather_call(tc, rid, pz_flat, flags):
    return _make_gather()(tc, rid, pz_flat, flags)


def _reduce_body(g_ref, wm_ref, fv_ref, fw_ref):
    wm = wm_ref[...]
    tv = jnp.where(wm != 0.0, g_ref[0:8, :], 0.0)
    wv = jnp.where(wm != 0.0, g_ref[8:16, :], 0.0)
    fv_ref[...] = jnp.sum(tv * wm, axis=0, keepdims=True)
    fw_ref[...] = jnp.sum(wv * wm, axis=0, keepdims=True)


_reduce_call = pl.pallas_call(
    _reduce_body,
    grid=(_GRID,),
    in_specs=[pl.BlockSpec((16, _BLK), lambda i: (jnp.int32(0), i)),
              pl.BlockSpec((8, _BLK), lambda i: (jnp.int32(0), i))],
    out_specs=[pl.BlockSpec((1, _BLK), lambda i: (jnp.int32(0), i))] * 2,
    out_shape=[jax.ShapeDtypeStruct((1, _P), jnp.float32)] * 2,
)


def kernel(depth, extrinsics, intrinsics, tsdf_volume, origin, resolution,
           weights_volume):
    b, h, w = depth.shape
    n = h * w
    coords = _world_points(depth, extrinsics, intrinsics)
    eye = extrinsics[:, :3, 3]
    ray_pts = _rays(coords, eye, origin, resolution)

    pts_t = jnp.transpose(ray_pts.reshape(_P, 3))
    w8, ixf, iyf, izf, rid, pz, wm = _trilinear_call(pts_t)

    flags = ((wm != 0.0).any(axis=0).reshape(_NW, _NCHK, _CH).any(axis=2)
             .astype(jnp.int32))
    flags = jnp.pad(flags, ((0, 0), (0, _FROW - _NCHK)))

    zpad = jnp.zeros((16,), jnp.float32)
    tp = jnp.concatenate([tsdf_volume.reshape(-1), zpad])
    wp = jnp.concatenate([weights_volume.reshape(-1), zpad])
    tc = jnp.concatenate(
        [tp[:_VN].reshape(_TROWS, 8), tp[8:_VN + 8].reshape(_TROWS, 8),
         wp[:_VN].reshape(_TROWS, 8), wp[8:_VN + 8].reshape(_TROWS, 8)],
        axis=1)

    g = _gather_call(tc, rid, pz.reshape(-1), flags)
    fv, fw = _reduce_call(g, wm)

    fusion_values = fv.reshape(b, _R, n)
    fusion_weights = fw.reshape(b, _R, n)
    weights_out = jnp.transpose(w8).reshape(b, _R, n, 8)
    idxf = jnp.stack([ixf, iyf, izf], axis=-1)
    indices = (jnp.transpose(idxf, (1, 0, 2))
               .astype(jnp.int64).reshape(b, _R, n, 8, 3))
    return (fusion_values, fusion_weights, ray_pts, depth.reshape(b, n),
            indices, weights_out, coords)
